# Initial kernel scaffold; baseline (speedup 1.0000x reference)
import functools

import jax
import jax.numpy as jnp
from jax import lax
from jax.experimental import pallas as pl
from jax.experimental.pallas import tpu as pltpu
from jax.experimental.pallas import tpu_sc as plsc

B = 16384
D = 64
MARGIN = 9.0
EMB_RANGE = (9.0 + 2.0) / 64.0
PHASE_SCALE = 3.141592653589793 / EMB_RANGE

NC = 2
NS = 16
L = 16
NW = NC * NS
PER_W = B // NW
CHUNK = 256
NCHUNK = PER_W // CHUNK
NG = CHUNK // L

_COS_C = (1.0, -0.5, 0.0416666679084301, -0.0013888889225199819,
          2.4801576728350483e-05, -2.7556734494282864e-07,
          2.08656536493379e-09, -1.1355099152621229e-11,
          4.127407576414062e-14)
_SIN_C = (1.0, -0.1666666716337204, 0.008333333767950535,
          -0.0001984127302421257, 2.755734840320656e-06,
          -2.5052040442119505e-08, 1.6054611806648467e-10,
          -7.591362976601401e-13, 2.4842502255079286e-15)


def _poly_even(coeffs, t):
    acc = jnp.full((L,), coeffs[-1], jnp.float32)
    for c in coeffs[-2::-1]:
        acc = acc * t + jnp.float32(c)
    return acc


def _sqrt(x):
    bits = lax.bitcast_convert_type(x, jnp.int32)
    seed = jnp.int32(0x5F3759DF) - lax.shift_right_logical(bits, 1)
    r = lax.bitcast_convert_type(seed, jnp.float32)
    for _ in range(3):
        r = r * (jnp.float32(1.5) - jnp.float32(0.5) * x * r * r)
    return x * r


_mesh = plsc.VectorSubcoreMesh(core_axis_name="c", subcore_axis_name="s")


@functools.partial(
    pl.kernel,
    out_type=jax.ShapeDtypeStruct((B,), jnp.float32),
    mesh=_mesh,
    scratch_types=[
        pltpu.VMEM((PER_W,), jnp.int32),
        pltpu.VMEM((PER_W,), jnp.int32),
        pltpu.VMEM((PER_W,), jnp.int32),
        pltpu.VMEM((CHUNK, 2 * D), jnp.float32),
        pltpu.VMEM((CHUNK, D), jnp.float32),
        pltpu.VMEM((CHUNK, 2 * D), jnp.float32),
        pltpu.VMEM((PER_W,), jnp.float32),
        pltpu.SemaphoreType.DMA,
    ],
)
def _rotate_score(h_hbm, r_hbm, t_hbm, ent_hbm, rel_hbm, out_hbm,
                  h_idx, r_idx, t_idx, h_rows, r_rows, t_rows, out_v, sem):
    wid = lax.axis_index("s") * NC + lax.axis_index("c")
    base = wid * PER_W
    pltpu.sync_copy(h_hbm.at[pl.ds(base, PER_W)], h_idx)
    pltpu.sync_copy(r_hbm.at[pl.ds(base, PER_W)], r_idx)
    pltpu.sync_copy(t_hbm.at[pl.ds(base, PER_W)], t_idx)

    for ci in range(NCHUNK):
        off = ci * CHUNK
        cp_h = pltpu.async_copy(
            ent_hbm.at[h_idx.at[pl.ds(off, CHUNK)]], h_rows, sem)
        cp_r = pltpu.async_copy(
            rel_hbm.at[r_idx.at[pl.ds(off, CHUNK)]], r_rows, sem)
        cp_t = pltpu.async_copy(
            ent_hbm.at[t_idx.at[pl.ds(off, CHUNK)]], t_rows, sem)
        cp_h.wait()
        cp_r.wait()
        cp_t.wait()

        def group_body(g, carry, off=off):
            rows = g * L + lax.iota(jnp.int32, L)

            def d_body(d, acc):
                dcol = jnp.full((L,), d, jnp.int32)
                re_h = plsc.load_gather(h_rows, [rows, dcol])
                im_h = plsc.load_gather(h_rows, [rows, dcol + D])
                re_t = plsc.load_gather(t_rows, [rows, dcol])
                im_t = plsc.load_gather(t_rows, [rows, dcol + D])
                ph = plsc.load_gather(r_rows, [rows, dcol]) * jnp.float32(
                    PHASE_SCALE)
                t2 = ph * ph
                cr = _poly_even(_COS_C, t2)
                sr = ph * _poly_even(_SIN_C, t2)
                dx = re_h * cr - im_h * sr - re_t
                dy = re_h * sr + im_h * cr - im_t
                return acc + _sqrt(dx * dx + dy * dy)

            acc = lax.fori_loop(0, D, d_body, jnp.zeros((L,), jnp.float32))
            out_v[pl.ds(off + g * L, L)] = jnp.float32(MARGIN) - acc
            return carry

        lax.fori_loop(0, NG, group_body, 0)

    pltpu.sync_copy(out_v, out_hbm.at[pl.ds(base, PER_W)])


def kernel(h, r, t, entity_embedding, relation_embedding):
    return _rotate_score(h.astype(jnp.int32), r.astype(jnp.int32),
                         t.astype(jnp.int32), entity_embedding,
                         relation_embedding)


# speedup vs baseline: 2.5954x; 2.5954x over previous
import functools

import jax
import jax.numpy as jnp
from jax import lax
from jax.experimental import pallas as pl
from jax.experimental.pallas import tpu as pltpu
from jax.experimental.pallas import tpu_sc as plsc

B = 16384
D = 64
MARGIN = 9.0
EMB_RANGE = (9.0 + 2.0) / 64.0
PHASE_SCALE = 3.141592653589793 / EMB_RANGE

NC = 2
NS = 16
L = 16
NW = NC * NS
PER_W = B // NW
CHUNK = 256
NCHUNK = PER_W // CHUNK
NG = CHUNK // L

_COS_C = (1.0, -0.5, 0.0416666679084301, -0.0013888889225199819,
          2.4801576728350483e-05, -2.7556734494282864e-07,
          2.08656536493379e-09, -1.1355099152621229e-11,
          4.127407576414062e-14)
_SIN_C = (1.0, -0.1666666716337204, 0.008333333767950535,
          -0.0001984127302421257, 2.755734840320656e-06,
          -2.5052040442119505e-08, 1.6054611806648467e-10,
          -7.591362976601401e-13, 2.4842502255079286e-15)


def _poly_even(coeffs, t):
    acc = jnp.full((L,), coeffs[-1], jnp.float32)
    for c in coeffs[-2::-1]:
        acc = acc * t + jnp.float32(c)
    return acc


def _sqrt(x):
    bits = lax.bitcast_convert_type(x, jnp.int32)
    seed = jnp.int32(0x5F3759DF) - lax.shift_right_logical(bits, 1)
    r = lax.bitcast_convert_type(seed, jnp.float32)
    for _ in range(3):
        r = r * (jnp.float32(1.5) - jnp.float32(0.5) * x * r * r)
    return x * r


_mesh = plsc.VectorSubcoreMesh(core_axis_name="c", subcore_axis_name="s")


@functools.partial(
    pl.kernel,
    out_type=jax.ShapeDtypeStruct((B,), jnp.float32),
    mesh=_mesh,
    compiler_params=pltpu.CompilerParams(needs_layout_passes=False),
    scratch_types=[
        pltpu.VMEM((PER_W,), jnp.int32),
        pltpu.VMEM((PER_W,), jnp.int32),
        pltpu.VMEM((PER_W,), jnp.int32),
        pltpu.VMEM((CHUNK, 2 * D), jnp.float32),
        pltpu.VMEM((CHUNK, 2 * D), jnp.float32),
        pltpu.VMEM((CHUNK, 2 * D), jnp.float32),
        pltpu.VMEM((PER_W,), jnp.float32),
        pltpu.SemaphoreType.DMA,
    ],
)
def _rotate_score(h_hbm, r_hbm, t_hbm, ent_hbm, rel_hbm, out_hbm,
                  h_idx, r_idx, t_idx, h_rows, r_rows, t_rows, out_v, sem):
    wid = lax.axis_index("s") * NC + lax.axis_index("c")
    base = wid * PER_W
    pltpu.sync_copy(h_hbm.at[pl.ds(base, PER_W)], h_idx)
    pltpu.sync_copy(r_hbm.at[pl.ds(base, PER_W)], r_idx)
    pltpu.sync_copy(t_hbm.at[pl.ds(base, PER_W)], t_idx)

    for ci in range(NCHUNK):
        off = ci * CHUNK
        cp_h = pltpu.async_copy(
            ent_hbm.at[h_idx.at[pl.ds(off, CHUNK)]], h_rows, sem)
        cp_r = pltpu.async_copy(
            rel_hbm.at[r_idx.at[pl.ds(off, CHUNK)]], r_rows, sem)
        cp_t = pltpu.async_copy(
            ent_hbm.at[t_idx.at[pl.ds(off, CHUNK)]], t_rows, sem)
        cp_h.wait()
        cp_r.wait()
        cp_t.wait()

        lanes = lax.iota(jnp.int32, L)

        def group_body(g, carry, off=off):
            def triple_body(p, out_acc):
                c = g * L + p
                acc = jnp.zeros((L,), jnp.float32)
                for j in range(D // L):
                    re_h = h_rows[c, pl.ds(j * L, L)]
                    im_h = h_rows[c, pl.ds(D + j * L, L)]
                    re_t = t_rows[c, pl.ds(j * L, L)]
                    im_t = t_rows[c, pl.ds(D + j * L, L)]
                    ph = r_rows[c, pl.ds(j * L, L)] * jnp.float32(PHASE_SCALE)
                    t2 = ph * ph
                    cr = _poly_even(_COS_C, t2)
                    sr = ph * _poly_even(_SIN_C, t2)
                    dx = re_h * cr - im_h * sr - re_t
                    dy = re_h * sr + im_h * cr - im_t
                    acc = acc + _sqrt(dx * dx + dy * dy)
                total = jnp.sum(acc)
                mask = (lanes == p).astype(jnp.float32)
                return out_acc + jnp.full((L,), total, jnp.float32) * mask

            out_acc = lax.fori_loop(0, L, triple_body,
                                    jnp.zeros((L,), jnp.float32))
            out_v[pl.ds(off + g * L, L)] = jnp.float32(MARGIN) - out_acc
            return carry

        lax.fori_loop(0, NG, group_body, 0)

    pltpu.sync_copy(out_v, out_hbm.at[pl.ds(base, PER_W)])


def kernel(h, r, t, entity_embedding, relation_embedding):
    rel = jnp.pad(relation_embedding, ((0, 0), (0, D)))
    return _rotate_score(h.astype(jnp.int32), r.astype(jnp.int32),
                         t.astype(jnp.int32), entity_embedding, rel)


# speedup vs baseline: 3.1984x; 1.2323x over previous
import functools

import jax
import jax.numpy as jnp
from jax import lax
from jax.experimental import pallas as pl
from jax.experimental.pallas import tpu as pltpu
from jax.experimental.pallas import tpu_sc as plsc

B = 16384
D = 64
MARGIN = 9.0
EMB_RANGE = (9.0 + 2.0) / 64.0
PHASE_SCALE = 3.141592653589793 / EMB_RANGE

NC = 2
NS = 16
L = 16
NW = NC * NS
PER_W = B // NW
CHUNK = 256
NCHUNK = PER_W // CHUNK
NG = CHUNK // L

NUM_REL = 1000


def _trig_body(rel_ref, out_ref):
    ph = rel_ref[...] * jnp.float32(PHASE_SCALE)
    out_ref[:, :D] = jnp.cos(ph)
    out_ref[:, D:] = jnp.sin(ph)


_trig_table = pl.pallas_call(
    _trig_body,
    out_shape=jax.ShapeDtypeStruct((NUM_REL, 2 * D), jnp.float32),
)


def _sqrt(x):
    bits = lax.bitcast_convert_type(x, jnp.int32)
    seed = jnp.int32(0x5F3759DF) - lax.shift_right_logical(bits, 1)
    r = lax.bitcast_convert_type(seed, jnp.float32)
    for _ in range(3):
        r = r * (jnp.float32(1.5) - jnp.float32(0.5) * x * r * r)
    return x * r


_mesh = plsc.VectorSubcoreMesh(core_axis_name="c", subcore_axis_name="s")


@functools.partial(
    pl.kernel,
    out_type=jax.ShapeDtypeStruct((B,), jnp.float32),
    mesh=_mesh,
    compiler_params=pltpu.CompilerParams(needs_layout_passes=False),
    scratch_types=[
        pltpu.VMEM((PER_W,), jnp.int32),
        pltpu.VMEM((PER_W,), jnp.int32),
        pltpu.VMEM((PER_W,), jnp.int32),
        pltpu.VMEM((CHUNK, 2 * D), jnp.float32),
        pltpu.VMEM((CHUNK, 2 * D), jnp.float32),
        pltpu.VMEM((CHUNK, 2 * D), jnp.float32),
        pltpu.VMEM((PER_W,), jnp.float32),
        pltpu.SemaphoreType.DMA,
    ],
)
def _rotate_score(h_hbm, r_hbm, t_hbm, ent_hbm, rel_hbm, out_hbm,
                  h_idx, r_idx, t_idx, h_rows, r_rows, t_rows, out_v, sem):
    wid = lax.axis_index("s") * NC + lax.axis_index("c")
    base = wid * PER_W
    pltpu.sync_copy(h_hbm.at[pl.ds(base, PER_W)], h_idx)
    pltpu.sync_copy(r_hbm.at[pl.ds(base, PER_W)], r_idx)
    pltpu.sync_copy(t_hbm.at[pl.ds(base, PER_W)], t_idx)

    for ci in range(NCHUNK):
        off = ci * CHUNK
        cp_h = pltpu.async_copy(
            ent_hbm.at[h_idx.at[pl.ds(off, CHUNK)]], h_rows, sem)
        cp_r = pltpu.async_copy(
            rel_hbm.at[r_idx.at[pl.ds(off, CHUNK)]], r_rows, sem)
        cp_t = pltpu.async_copy(
            ent_hbm.at[t_idx.at[pl.ds(off, CHUNK)]], t_rows, sem)
        cp_h.wait()
        cp_r.wait()
        cp_t.wait()

        lanes = lax.iota(jnp.int32, L)

        def group_body(g, carry, off=off):
            def triple_body(p, out_acc):
                c = g * L + p
                acc = jnp.zeros((L,), jnp.float32)
                for j in range(D // L):
                    re_h = h_rows[c, pl.ds(j * L, L)]
                    im_h = h_rows[c, pl.ds(D + j * L, L)]
                    re_t = t_rows[c, pl.ds(j * L, L)]
                    im_t = t_rows[c, pl.ds(D + j * L, L)]
                    cr = r_rows[c, pl.ds(j * L, L)]
                    sr = r_rows[c, pl.ds(D + j * L, L)]
                    dx = re_h * cr - im_h * sr - re_t
                    dy = re_h * sr + im_h * cr - im_t
                    acc = acc + _sqrt(dx * dx + dy * dy)
                total = jnp.sum(acc)
                mask = (lanes == p).astype(jnp.float32)
                return out_acc + jnp.full((L,), total, jnp.float32) * mask

            out_acc = lax.fori_loop(0, L, triple_body,
                                    jnp.zeros((L,), jnp.float32))
            out_v[pl.ds(off + g * L, L)] = jnp.float32(MARGIN) - out_acc
            return carry

        lax.fori_loop(0, NG, group_body, 0)

    pltpu.sync_copy(out_v, out_hbm.at[pl.ds(base, PER_W)])


def kernel(h, r, t, entity_embedding, relation_embedding):
    rel_cs = _trig_table(relation_embedding)
    return _rotate_score(h.astype(jnp.int32), r.astype(jnp.int32),
                         t.astype(jnp.int32), entity_embedding, rel_cs)


# speedup vs baseline: 3.7066x; 1.1589x over previous
import functools

import jax
import jax.numpy as jnp
from jax import lax
from jax.experimental import pallas as pl
from jax.experimental.pallas import tpu as pltpu
from jax.experimental.pallas import tpu_sc as plsc

B = 16384
D = 64
MARGIN = 9.0
EMB_RANGE = (9.0 + 2.0) / 64.0
PHASE_SCALE = 3.141592653589793 / EMB_RANGE

NC = 2
NS = 16
L = 16
NW = NC * NS
PER_W = B // NW
CHUNK = 128
NCHUNK = PER_W // CHUNK
NBUF = 2
NG = CHUNK // L

NUM_REL = 1000


def _trig_body(rel_ref, out_ref):
    ph = rel_ref[...] * jnp.float32(PHASE_SCALE)
    out_ref[:, :D] = jnp.cos(ph)
    out_ref[:, D:] = jnp.sin(ph)


_trig_table = pl.pallas_call(
    _trig_body,
    out_shape=jax.ShapeDtypeStruct((NUM_REL, 2 * D), jnp.float32),
)


def _sqrt(x):
    bits = lax.bitcast_convert_type(x, jnp.int32)
    seed = jnp.int32(0x5F3759DF) - lax.shift_right_logical(bits, 1)
    r = lax.bitcast_convert_type(seed, jnp.float32)
    for _ in range(2):
        r = r * (jnp.float32(1.5) - jnp.float32(0.5) * x * r * r)
    return x * r


_mesh = plsc.VectorSubcoreMesh(core_axis_name="c", subcore_axis_name="s")


@functools.partial(
    pl.kernel,
    out_type=jax.ShapeDtypeStruct((B,), jnp.float32),
    mesh=_mesh,
    compiler_params=pltpu.CompilerParams(needs_layout_passes=False),
    scratch_types=[
        pltpu.VMEM((PER_W,), jnp.int32),
        pltpu.VMEM((PER_W,), jnp.int32),
        pltpu.VMEM((PER_W,), jnp.int32),
        [pltpu.VMEM((CHUNK, 2 * D), jnp.float32) for _ in range(NBUF)],
        [pltpu.VMEM((CHUNK, 2 * D), jnp.float32) for _ in range(NBUF)],
        [pltpu.VMEM((CHUNK, 2 * D), jnp.float32) for _ in range(NBUF)],
        pltpu.VMEM((PER_W,), jnp.float32),
        [pltpu.SemaphoreType.DMA for _ in range(NBUF)],
    ],
)
def _rotate_score(h_hbm, r_hbm, t_hbm, ent_hbm, rel_hbm, out_hbm,
                  h_idx, r_idx, t_idx, h_bufs, r_bufs, t_bufs, out_v, sems):
    wid = lax.axis_index("s") * NC + lax.axis_index("c")
    base = wid * PER_W
    pltpu.sync_copy(h_hbm.at[pl.ds(base, PER_W)], h_idx)
    pltpu.sync_copy(r_hbm.at[pl.ds(base, PER_W)], r_idx)
    pltpu.sync_copy(t_hbm.at[pl.ds(base, PER_W)], t_idx)

    lanes = lax.iota(jnp.int32, L)

    def start(ci):
        sl = ci % NBUF
        off = ci * CHUNK
        return [
            pltpu.async_copy(
                ent_hbm.at[h_idx.at[pl.ds(off, CHUNK)]], h_bufs[sl], sems[sl]),
            pltpu.async_copy(
                rel_hbm.at[r_idx.at[pl.ds(off, CHUNK)]], r_bufs[sl], sems[sl]),
            pltpu.async_copy(
                ent_hbm.at[t_idx.at[pl.ds(off, CHUNK)]], t_bufs[sl], sems[sl]),
        ]

    def compute(ci):
        sl = ci % NBUF
        off = ci * CHUNK
        h_rows, r_rows, t_rows = h_bufs[sl], r_bufs[sl], t_bufs[sl]

        def group_body(g, carry):
            def triple_body(p, out_acc):
                c = g * L + p
                acc = jnp.zeros((L,), jnp.float32)
                for j in range(D // L):
                    re_h = h_rows[c, pl.ds(j * L, L)]
                    im_h = h_rows[c, pl.ds(D + j * L, L)]
                    re_t = t_rows[c, pl.ds(j * L, L)]
                    im_t = t_rows[c, pl.ds(D + j * L, L)]
                    cr = r_rows[c, pl.ds(j * L, L)]
                    sr = r_rows[c, pl.ds(D + j * L, L)]
                    dx = re_h * cr - im_h * sr - re_t
                    dy = re_h * sr + im_h * cr - im_t
                    acc = acc + _sqrt(dx * dx + dy * dy)
                total = jnp.sum(acc)
                mask = (lanes == p).astype(jnp.float32)
                return out_acc + jnp.full((L,), total, jnp.float32) * mask

            out_acc = lax.fori_loop(0, L, triple_body,
                                    jnp.zeros((L,), jnp.float32))
            out_v[pl.ds(off + g * L, L)] = jnp.float32(MARGIN) - out_acc
            return carry

        lax.fori_loop(0, NG, group_body, 0)

    pending = start(0)
    for ci in range(NCHUNK):
        nxt = start(ci + 1) if ci + 1 < NCHUNK else []
        for cp in pending:
            cp.wait()
        pending = nxt
        compute(ci)

    pltpu.sync_copy(out_v, out_hbm.at[pl.ds(base, PER_W)])


def kernel(h, r, t, entity_embedding, relation_embedding):
    rel_cs = _trig_table(relation_embedding)
    return _rotate_score(h.astype(jnp.int32), r.astype(jnp.int32),
                         t.astype(jnp.int32), entity_embedding, rel_cs)
